# Initial kernel scaffold; baseline (speedup 1.0000x reference)
import jax
import jax.numpy as jnp
from jax.experimental import pallas as pl


def kernel(inputs, edge_index, W0, W1, W2):
    raise NotImplementedError("write your pallas kernel here")


# speedup vs baseline: 4.6765x; 4.6765x over previous
import functools

import jax
import jax.numpy as jnp
from jax import lax
from jax.experimental import pallas as pl
from jax.experimental.pallas import tpu as pltpu
from jax.experimental.pallas import tpu_sc as plsc

N = 10000
E = 160000
IN = 256
HID = 512
CLS = 64

N_PAD = 10240
E_PAD = 163840
G = 128
NSC = 2
NT = 16
ROWS_PER_TILE = N_PAD // NT
RB = 2000

_mesh = plsc.VectorSubcoreMesh(core_axis_name="c", subcore_axis_name="s")
_SC_PARAMS = pltpu.CompilerParams(use_tc_tiling_on_sc=False)


def _zero_fill(buf, width):
    zv = jnp.zeros((16,), jnp.float32)

    def row(g, _):
        for i in range(width // 16):
            buf[g, pl.ds(i * 16, 16)] = zv
        return 0

    lax.fori_loop(0, 128, row, 0)


def _deg_sc(dst2):
    gpt = E_PAD // (NSC * NT) // G

    @functools.partial(
        pl.kernel,
        out_type=jax.ShapeDtypeStruct((NSC, N_PAD, 16), jnp.float32),
        mesh=_mesh,
        compiler_params=_SC_PARAMS,
        scratch_types=[
            pltpu.VMEM((gpt, G), jnp.int32),
            pltpu.VMEM((G, 16), jnp.float32),
            pltpu.VMEM((G, 16), jnp.float32),
            pltpu.VMEM_SHARED((N_PAD, 16), jnp.float32),
        ],
    )
    def kern(dst2_hbm, out_hbm, dst_sl, ones_b, tmp_b, acc):
        cid = lax.axis_index("c")
        sid = lax.axis_index("s")
        wid = cid * NT + sid
        pltpu.sync_copy(dst2_hbm.at[pl.ds(wid * gpt, gpt)], dst_sl)

        ov = jnp.ones((16,), jnp.float32)

        def fill(g, _):
            ones_b[g] = ov
            return 0

        lax.fori_loop(0, G, fill, 0)
        _zero_fill(tmp_b, 16)

        base = sid * ROWS_PER_TILE
        for b in range(ROWS_PER_TILE // G):
            pltpu.sync_copy(tmp_b, acc.at[pl.ds(base + b * G, G)])
        plsc.subcore_barrier()

        def sloop(g, _):
            pltpu.sync_copy(ones_b, acc.at[dst_sl.at[g]], add=True)
            return 0

        lax.fori_loop(0, gpt, sloop, 0)
        plsc.subcore_barrier()

        for b in range(ROWS_PER_TILE // G):
            r = base + b * G
            pltpu.sync_copy(acc.at[pl.ds(r, G)], tmp_b)
            pltpu.sync_copy(tmp_b, out_hbm.at[cid].at[pl.ds(r, G)])

    return kern(dst2)


def _propagate_sc(hs2, src2, dst2, nc, width, split_edges):
    if split_edges:
        assert nc == 1
        n_out = NSC
        chunks_per_sc = 1
        gpt = E_PAD // 2 // NT // G
    else:
        n_out = nc
        chunks_per_sc = nc // NSC
        gpt = E_PAD // NT // G

    @functools.partial(
        pl.kernel,
        out_type=jax.ShapeDtypeStruct((n_out, N_PAD, width), jnp.float32),
        mesh=_mesh,
        compiler_params=_SC_PARAMS,
        scratch_types=[
            pltpu.VMEM((gpt, G), jnp.int32),
            pltpu.VMEM((gpt, G), jnp.int32),
            pltpu.VMEM((gpt, G), jnp.int32),
            pltpu.VMEM((G, width), jnp.float32),
            pltpu.VMEM((G, width), jnp.float32),
            pltpu.VMEM_SHARED((N_PAD, width), jnp.float32),
            pltpu.SemaphoreType.DMA,
        ],
    )
    def kern(hs_hbm, src2_hbm, dst2_hbm, out_hbm,
             src_sl, dst_sl, gidx, rows, tmp_b, acc, sem):
        cid = lax.axis_index("c")
        sid = lax.axis_index("s")
        if split_edges:
            row0 = (cid * NT + sid) * gpt
        else:
            row0 = sid * gpt
        pltpu.sync_copy(src2_hbm.at[pl.ds(row0, gpt)], src_sl)
        pltpu.sync_copy(dst2_hbm.at[pl.ds(row0, gpt)], dst_sl)

        _zero_fill(tmp_b, width)
        base = sid * ROWS_PER_TILE

        def zero_acc():
            for b in range(ROWS_PER_TILE // G):
                pltpu.sync_copy(tmp_b, acc.at[pl.ds(base + b * G, G)])

        zero_acc()
        plsc.subcore_barrier()

        for k in range(chunks_per_sc):
            chunk = cid * chunks_per_sc + k
            out_slot = cid if split_edges else chunk

            if nc == 1:
                idx_ref = src_sl
            else:
                idx_ref = gidx

                def bidx(g, _):
                    for i in range(G // 16):
                        v = src_sl[g, pl.ds(i * 16, 16)]
                        gidx[g, pl.ds(i * 16, 16)] = v * nc + chunk
                    return 0

                lax.fori_loop(0, gpt, bidx, 0)

            def sloop(g, _):
                pltpu.async_copy(hs_hbm.at[idx_ref.at[g]], rows, sem).wait()
                pltpu.sync_copy(rows, acc.at[dst_sl.at[g]], add=True)
                return 0

            lax.fori_loop(0, gpt, sloop, 0)
            plsc.subcore_barrier()

            for b in range(ROWS_PER_TILE // G):
                r = base + b * G
                pltpu.sync_copy(acc.at[pl.ds(r, G)], rows)
                pltpu.sync_copy(rows, out_hbm.at[out_slot].at[pl.ds(r, G)])

            if k + 1 < chunks_per_sc:
                plsc.subcore_barrier()
                zero_acc()
                plsc.subcore_barrier()

    return kern(hs2, src2, dst2)


def _dinv_of(deg_ref):
    d = deg_ref[0, :, 0:1] + deg_ref[1, :, 0:1]
    return lax.rsqrt(jnp.maximum(d, 1.0))


def _pre_body(deg_ref, x_ref, out_ref):
    out_ref[...] = x_ref[...] * _dinv_of(deg_ref)


def _mm1_body(deg_ref, p_ref, w_ref, out_ref):
    dinv = _dinv_of(deg_ref)
    t = jnp.zeros((RB, HID), jnp.float32)
    for c in range(IN // 64):
        t += jnp.dot(p_ref[c], w_ref[pl.ds(c * 64, 64), :],
                     preferred_element_type=jnp.float32)
    out_ref[...] = dinv * jax.nn.relu(dinv * t)


def _mm23_body(deg_ref, p_ref, w1_ref, w2_ref, out_ref):
    dinv = _dinv_of(deg_ref)
    t = jnp.zeros((RB, HID), jnp.float32)
    for c in range(HID // 64):
        t += jnp.dot(p_ref[c], w1_ref[pl.ds(c * 64, 64), :],
                     preferred_element_type=jnp.float32)
    h2 = jax.nn.relu(dinv * t)
    out_ref[...] = dinv * jnp.dot(h2, w2_ref[...],
                                  preferred_element_type=jnp.float32)


def _final_body(deg_ref, p_ref, out_ref):
    dinv = _dinv_of(deg_ref)
    y = dinv * (p_ref[0] + p_ref[1])
    m = jnp.max(y, axis=1, keepdims=True)
    e = jnp.exp(y - m)
    out_ref[...] = (y - m) - jnp.log(jnp.sum(e, axis=1, keepdims=True))


def _deg_spec():
    return pl.BlockSpec((2, RB, 16), lambda i: (0, i, 0))


def _chunk_spec(nc, width=128):
    return pl.BlockSpec((nc, RB, width), lambda i: (0, i, 0))


def _full_spec(shape):
    return pl.BlockSpec(shape, lambda i: tuple(0 for _ in shape))


def _row_spec(width):
    return pl.BlockSpec((RB, width), lambda i: (i, 0))


_GRID = (N // RB,)


def _pre_tc(deg2, x):
    return pl.pallas_call(
        _pre_body,
        grid=_GRID,
        in_specs=[_deg_spec(), _row_spec(IN)],
        out_specs=_row_spec(IN),
        out_shape=jax.ShapeDtypeStruct((N, IN), jnp.float32),
    )(deg2, x)


def _mm1_tc(deg2, p1, w0):
    return pl.pallas_call(
        _mm1_body,
        grid=_GRID,
        in_specs=[_deg_spec(), _chunk_spec(4, 64), _full_spec((IN, HID))],
        out_specs=_row_spec(HID),
        out_shape=jax.ShapeDtypeStruct((N, HID), jnp.float32),
    )(deg2, p1, w0)


def _mm23_tc(deg2, p2, w1, w2):
    return pl.pallas_call(
        _mm23_body,
        grid=_GRID,
        in_specs=[_deg_spec(), _chunk_spec(8, 64), _full_spec((HID, HID)),
                  _full_spec((HID, CLS))],
        out_specs=_row_spec(CLS),
        out_shape=jax.ShapeDtypeStruct((N, CLS), jnp.float32),
    )(deg2, p2, w1, w2)


def _final_tc(deg2, p3):
    return pl.pallas_call(
        _final_body,
        grid=_GRID,
        in_specs=[_deg_spec(), _chunk_spec(2, CLS)],
        out_specs=_row_spec(CLS),
        out_shape=jax.ShapeDtypeStruct((N, CLS), jnp.float32),
    )(deg2, p3)


def kernel(inputs, edge_index, W0, W1, W2):
    src = edge_index[0]
    dst = edge_index[1]
    pad = E_PAD - E
    src_p = jnp.concatenate([src, jnp.zeros((pad,), jnp.int32)])
    dst_p = jnp.concatenate([dst, jnp.full((pad,), N, jnp.int32)])
    src2 = src_p.reshape(E_PAD // G, G)
    dst2 = dst_p.reshape(E_PAD // G, G)

    deg2 = _deg_sc(dst2)
    xs = _pre_tc(deg2, inputs)
    p1 = _propagate_sc(xs.reshape(N * 4, 64), src2, dst2,
                       nc=4, width=64, split_edges=False)
    hs1 = _mm1_tc(deg2, p1, W0)
    p2 = _propagate_sc(hs1.reshape(N * 8, 64), src2, dst2,
                       nc=8, width=64, split_edges=False)
    g = _mm23_tc(deg2, p2, W1, W2)
    p3 = _propagate_sc(g, src2, dst2,
                       nc=1, width=CLS, split_edges=True)
    return _final_tc(deg2, p3)


# speedup vs baseline: 5.9081x; 1.2634x over previous
import functools

import jax
import jax.numpy as jnp
from jax import lax
from jax.experimental import pallas as pl
from jax.experimental.pallas import tpu as pltpu
from jax.experimental.pallas import tpu_sc as plsc

N = 10000
E = 160000
IN = 256
HID = 512
CLS = 64

N_PAD = 10240
E_PAD = 163840
G = 128
NSC = 2
NT = 16
ROWS_PER_TILE = N_PAD // NT
RB = 2000

_mesh = plsc.VectorSubcoreMesh(core_axis_name="c", subcore_axis_name="s")
_SC_PARAMS = pltpu.CompilerParams(use_tc_tiling_on_sc=False)


def _zero_fill(buf, width):
    zv = jnp.zeros((16,), jnp.float32)

    def row(g, _):
        for i in range(width // 16):
            buf[g, pl.ds(i * 16, 16)] = zv
        return 0

    lax.fori_loop(0, 128, row, 0)


def _deg_sc(dst2):
    gpt = E_PAD // (NSC * NT) // G

    @functools.partial(
        pl.kernel,
        out_type=jax.ShapeDtypeStruct((NSC, N_PAD, 16), jnp.float32),
        mesh=_mesh,
        compiler_params=_SC_PARAMS,
        scratch_types=[
            pltpu.VMEM((gpt, G), jnp.int32),
            pltpu.VMEM((G, 16), jnp.float32),
            pltpu.VMEM((G, 16), jnp.float32),
            pltpu.VMEM_SHARED((N_PAD, 16), jnp.float32),
        ],
    )
    def kern(dst2_hbm, out_hbm, dst_sl, ones_b, tmp_b, acc):
        cid = lax.axis_index("c")
        sid = lax.axis_index("s")
        wid = cid * NT + sid
        pltpu.sync_copy(dst2_hbm.at[pl.ds(wid * gpt, gpt)], dst_sl)

        ov = jnp.ones((16,), jnp.float32)

        def fill(g, _):
            ones_b[g] = ov
            return 0

        lax.fori_loop(0, G, fill, 0)
        _zero_fill(tmp_b, 16)

        base = sid * ROWS_PER_TILE
        for b in range(ROWS_PER_TILE // G):
            pltpu.sync_copy(tmp_b, acc.at[pl.ds(base + b * G, G)])
        plsc.subcore_barrier()

        def sloop(g, _):
            pltpu.sync_copy(ones_b, acc.at[dst_sl.at[g]], add=True)
            return 0

        lax.fori_loop(0, gpt, sloop, 0)
        plsc.subcore_barrier()

        for b in range(ROWS_PER_TILE // G):
            r = base + b * G
            pltpu.sync_copy(acc.at[pl.ds(r, G)], tmp_b)
            pltpu.sync_copy(tmp_b, out_hbm.at[cid].at[pl.ds(r, G)])

    return kern(dst2)


def _propagate_sc(hs2, src2, dst2, nc, width, split_edges):
    if split_edges:
        assert nc == 1
        n_out = NSC
        chunks_per_sc = 1
        gpt = E_PAD // 2 // NT // G
    else:
        n_out = nc
        chunks_per_sc = nc // NSC
        gpt = E_PAD // NT // G

    @functools.partial(
        pl.kernel,
        out_type=jax.ShapeDtypeStruct((n_out, N_PAD, width), jnp.float32),
        mesh=_mesh,
        compiler_params=_SC_PARAMS,
        scratch_types=[
            pltpu.VMEM((gpt, G), jnp.int32),
            pltpu.VMEM((gpt, G), jnp.int32),
            pltpu.VMEM((gpt, G), jnp.int32),
            pltpu.VMEM((2 * G, width), jnp.float32),
            pltpu.VMEM((G, width), jnp.float32),
            pltpu.VMEM_SHARED((N_PAD, width), jnp.float32),
            pltpu.SemaphoreType.DMA,
            pltpu.SemaphoreType.DMA,
        ],
    )
    def kern(hs_hbm, src2_hbm, dst2_hbm, out_hbm,
             src_sl, dst_sl, gidx, rows, tmp_b, acc, gsem, ssem):
        cid = lax.axis_index("c")
        sid = lax.axis_index("s")
        if split_edges:
            row0 = (cid * NT + sid) * gpt
        else:
            row0 = sid * gpt
        pltpu.sync_copy(src2_hbm.at[pl.ds(row0, gpt)], src_sl)
        pltpu.sync_copy(dst2_hbm.at[pl.ds(row0, gpt)], dst_sl)

        _zero_fill(tmp_b, width)
        base = sid * ROWS_PER_TILE

        def zero_acc():
            for b in range(ROWS_PER_TILE // G):
                pltpu.sync_copy(tmp_b, acc.at[pl.ds(base + b * G, G)])

        zero_acc()
        plsc.subcore_barrier()

        for k in range(chunks_per_sc):
            chunk = cid * chunks_per_sc + k
            out_slot = cid if split_edges else chunk

            if nc == 1:
                idx_ref = src_sl
            else:
                idx_ref = gidx

                def bidx(g, _):
                    for i in range(G // 16):
                        v = src_sl[g, pl.ds(i * 16, 16)]
                        gidx[g, pl.ds(i * 16, 16)] = v * nc + chunk
                    return 0

                lax.fori_loop(0, gpt, bidx, 0)

            def buf(par):
                return rows.at[pl.ds(par * G, G)]

            pltpu.async_copy(hs_hbm.at[idx_ref.at[0]], buf(0), gsem)

            def sloop(g, _):
                par = lax.bitwise_and(g, 1)
                nxt = 1 - par

                @pl.when(g >= 1)
                def _():
                    pltpu.make_async_copy(buf(nxt), acc.at[dst_sl.at[g - 1]],
                                          ssem).wait()

                @pl.when(g + 1 < gpt)
                def _():
                    pltpu.async_copy(hs_hbm.at[idx_ref.at[g + 1]], buf(nxt),
                                     gsem)

                pltpu.make_async_copy(hs_hbm.at[idx_ref.at[g]], buf(par),
                                      gsem).wait()
                pltpu.async_copy(buf(par), acc.at[dst_sl.at[g]], ssem,
                                 add=True)
                return 0

            lax.fori_loop(0, gpt, sloop, 0)
            pltpu.make_async_copy(buf(lax.bitwise_and(gpt - 1, 1)),
                                  acc.at[dst_sl.at[gpt - 1]], ssem).wait()
            plsc.subcore_barrier()

            for b in range(ROWS_PER_TILE // G):
                r = base + b * G
                pltpu.sync_copy(acc.at[pl.ds(r, G)], buf(0))
                pltpu.sync_copy(buf(0), out_hbm.at[out_slot].at[pl.ds(r, G)])

            if k + 1 < chunks_per_sc:
                plsc.subcore_barrier()
                zero_acc()
                plsc.subcore_barrier()

    return kern(hs2, src2, dst2)


def _dinv_of(deg_ref):
    d = deg_ref[0, :, 0:1] + deg_ref[1, :, 0:1]
    return lax.rsqrt(jnp.maximum(d, 1.0))


def _pre_body(deg_ref, x_ref, out_ref):
    out_ref[...] = x_ref[...] * _dinv_of(deg_ref)


def _mm1_body(deg_ref, p_ref, w_ref, out_ref):
    dinv = _dinv_of(deg_ref)
    t = jnp.zeros((RB, HID), jnp.float32)
    for c in range(IN // 64):
        t += jnp.dot(p_ref[c], w_ref[pl.ds(c * 64, 64), :],
                     preferred_element_type=jnp.float32)
    out_ref[...] = dinv * jax.nn.relu(dinv * t)


def _mm23_body(deg_ref, p_ref, w1_ref, w2_ref, out_ref):
    dinv = _dinv_of(deg_ref)
    t = jnp.zeros((RB, HID), jnp.float32)
    for c in range(HID // 64):
        t += jnp.dot(p_ref[c], w1_ref[pl.ds(c * 64, 64), :],
                     preferred_element_type=jnp.float32)
    h2 = jax.nn.relu(dinv * t)
    out_ref[...] = dinv * jnp.dot(h2, w2_ref[...],
                                  preferred_element_type=jnp.float32)


def _final_body(deg_ref, p_ref, out_ref):
    dinv = _dinv_of(deg_ref)
    y = dinv * (p_ref[0] + p_ref[1])
    m = jnp.max(y, axis=1, keepdims=True)
    e = jnp.exp(y - m)
    out_ref[...] = (y - m) - jnp.log(jnp.sum(e, axis=1, keepdims=True))


def _deg_spec():
    return pl.BlockSpec((2, RB, 16), lambda i: (0, i, 0))


def _chunk_spec(nc, width=128):
    return pl.BlockSpec((nc, RB, width), lambda i: (0, i, 0))


def _full_spec(shape):
    return pl.BlockSpec(shape, lambda i: tuple(0 for _ in shape))


def _row_spec(width):
    return pl.BlockSpec((RB, width), lambda i: (i, 0))


_GRID = (N // RB,)


def _pre_tc(deg2, x):
    return pl.pallas_call(
        _pre_body,
        grid=_GRID,
        in_specs=[_deg_spec(), _row_spec(IN)],
        out_specs=_row_spec(IN),
        out_shape=jax.ShapeDtypeStruct((N, IN), jnp.float32),
    )(deg2, x)


def _mm1_tc(deg2, p1, w0):
    return pl.pallas_call(
        _mm1_body,
        grid=_GRID,
        in_specs=[_deg_spec(), _chunk_spec(4, 64), _full_spec((IN, HID))],
        out_specs=_row_spec(HID),
        out_shape=jax.ShapeDtypeStruct((N, HID), jnp.float32),
    )(deg2, p1, w0)


def _mm23_tc(deg2, p2, w1, w2):
    return pl.pallas_call(
        _mm23_body,
        grid=_GRID,
        in_specs=[_deg_spec(), _chunk_spec(8, 64), _full_spec((HID, HID)),
                  _full_spec((HID, CLS))],
        out_specs=_row_spec(CLS),
        out_shape=jax.ShapeDtypeStruct((N, CLS), jnp.float32),
    )(deg2, p2, w1, w2)


def _final_tc(deg2, p3):
    return pl.pallas_call(
        _final_body,
        grid=_GRID,
        in_specs=[_deg_spec(), _chunk_spec(2, CLS)],
        out_specs=_row_spec(CLS),
        out_shape=jax.ShapeDtypeStruct((N, CLS), jnp.float32),
    )(deg2, p3)


def kernel(inputs, edge_index, W0, W1, W2):
    src = edge_index[0]
    dst = edge_index[1]
    pad = E_PAD - E
    src_p = jnp.concatenate([src, jnp.zeros((pad,), jnp.int32)])
    dst_p = jnp.concatenate([dst, jnp.full((pad,), N, jnp.int32)])
    src2 = src_p.reshape(E_PAD // G, G)
    dst2 = dst_p.reshape(E_PAD // G, G)

    deg2 = _deg_sc(dst2)
    xs = _pre_tc(deg2, inputs)
    p1 = _propagate_sc(xs.reshape(N * 4, 64), src2, dst2,
                       nc=4, width=64, split_edges=False)
    hs1 = _mm1_tc(deg2, p1, W0)
    p2 = _propagate_sc(hs1.reshape(N * 8, 64), src2, dst2,
                       nc=8, width=64, split_edges=False)
    g = _mm23_tc(deg2, p2, W1, W2)
    p3 = _propagate_sc(g, src2, dst2,
                       nc=1, width=CLS, split_edges=True)
    return _final_tc(deg2, p3)


# speedup vs baseline: 6.1810x; 1.0462x over previous
import functools

import jax
import jax.numpy as jnp
from jax import lax
from jax.experimental import pallas as pl
from jax.experimental.pallas import tpu as pltpu
from jax.experimental.pallas import tpu_sc as plsc

N = 10000
E = 160000
IN = 256
HID = 512
CLS = 64

N_PAD = 10240
E_PAD = 163840
G = 128
NSC = 2
NT = 16
ROWS_PER_TILE = N_PAD // NT
RB = 2000
RING = 4

_mesh = plsc.VectorSubcoreMesh(core_axis_name="c", subcore_axis_name="s")
_SC_PARAMS = pltpu.CompilerParams(use_tc_tiling_on_sc=False)


def _zero_fill(buf, width):
    zv = jnp.zeros((16,), jnp.float32)

    def row(g, _):
        for i in range(width // 16):
            buf[g, pl.ds(i * 16, 16)] = zv
        return 0

    lax.fori_loop(0, 128, row, 0)


def _deg_sc(dst2):
    gpt = E_PAD // (NSC * NT) // G

    @functools.partial(
        pl.kernel,
        out_type=jax.ShapeDtypeStruct((NSC, N_PAD, 16), jnp.float32),
        mesh=_mesh,
        compiler_params=_SC_PARAMS,
        scratch_types=[
            pltpu.VMEM((gpt, G), jnp.int32),
            pltpu.VMEM((G, 16), jnp.float32),
            pltpu.VMEM((G, 16), jnp.float32),
            pltpu.VMEM_SHARED((N_PAD, 16), jnp.float32),
        ],
    )
    def kern(dst2_hbm, out_hbm, dst_sl, ones_b, tmp_b, acc):
        cid = lax.axis_index("c")
        sid = lax.axis_index("s")
        wid = cid * NT + sid
        pltpu.sync_copy(dst2_hbm.at[pl.ds(wid * gpt, gpt)], dst_sl)

        ov = jnp.ones((16,), jnp.float32)

        def fill(g, _):
            ones_b[g] = ov
            return 0

        lax.fori_loop(0, G, fill, 0)
        _zero_fill(tmp_b, 16)

        base = sid * ROWS_PER_TILE
        for b in range(ROWS_PER_TILE // G):
            pltpu.sync_copy(tmp_b, acc.at[pl.ds(base + b * G, G)])
        plsc.subcore_barrier()

        def sloop(g, _):
            pltpu.sync_copy(ones_b, acc.at[dst_sl.at[g]], add=True)
            return 0

        lax.fori_loop(0, gpt, sloop, 0)
        plsc.subcore_barrier()

        pltpu.sync_copy(acc.at[pl.ds(base, ROWS_PER_TILE)],
                        out_hbm.at[cid].at[pl.ds(base, ROWS_PER_TILE)])

    return kern(dst2)


def _propagate_sc(hs2, src2, dst2, nc, width, split_edges):
    if split_edges:
        assert nc == 1
        n_out = NSC
        chunks_per_sc = 1
        gpt = E_PAD // 2 // NT // G
    else:
        n_out = nc
        chunks_per_sc = nc // NSC
        gpt = E_PAD // NT // G

    @functools.partial(
        pl.kernel,
        out_type=jax.ShapeDtypeStruct((n_out, N_PAD, width), jnp.float32),
        mesh=_mesh,
        compiler_params=_SC_PARAMS,
        scratch_types=[
            pltpu.VMEM((gpt, G), jnp.int32),
            pltpu.VMEM((gpt, G), jnp.int32),
            pltpu.VMEM((gpt, G), jnp.int32),
            pltpu.VMEM((RING * G, width), jnp.float32),
            pltpu.VMEM((G, width), jnp.float32),
            pltpu.VMEM_SHARED((N_PAD, width), jnp.float32),
            pltpu.SemaphoreType.DMA,
            pltpu.SemaphoreType.DMA,
        ],
    )
    def kern(hs_hbm, src2_hbm, dst2_hbm, out_hbm,
             src_sl, dst_sl, gidx, rows, tmp_b, acc, gsem, ssem):
        cid = lax.axis_index("c")
        sid = lax.axis_index("s")
        if split_edges:
            row0 = (cid * NT + sid) * gpt
        else:
            row0 = sid * gpt
        pltpu.sync_copy(src2_hbm.at[pl.ds(row0, gpt)], src_sl)
        pltpu.sync_copy(dst2_hbm.at[pl.ds(row0, gpt)], dst_sl)

        _zero_fill(tmp_b, width)
        base = sid * ROWS_PER_TILE

        def zero_acc():
            for b in range(ROWS_PER_TILE // G):
                pltpu.sync_copy(tmp_b, acc.at[pl.ds(base + b * G, G)])

        zero_acc()
        plsc.subcore_barrier()

        for k in range(chunks_per_sc):
            chunk = cid * chunks_per_sc + k
            out_slot = cid if split_edges else chunk

            if nc == 1:
                idx_ref = src_sl
            else:
                idx_ref = gidx

                def bidx(g, _):
                    for i in range(G // 16):
                        v = src_sl[g, pl.ds(i * 16, 16)]
                        gidx[g, pl.ds(i * 16, 16)] = v * nc + chunk
                    return 0

                lax.fori_loop(0, gpt, bidx, 0)

            def buf(par):
                return rows.at[pl.ds(par * G, G)]

            for i in range(RING - 1):
                pltpu.async_copy(hs_hbm.at[idx_ref.at[i]], buf(i), gsem)

            def sloop(g, _):
                par = lax.bitwise_and(g, RING - 1)
                nxt = lax.bitwise_and(g + RING - 1, RING - 1)

                @pl.when(g >= 1)
                def _():
                    pltpu.make_async_copy(buf(nxt), acc.at[dst_sl.at[g - 1]],
                                          ssem).wait()

                @pl.when(g + RING - 1 < gpt)
                def _():
                    pltpu.async_copy(hs_hbm.at[idx_ref.at[g + RING - 1]],
                                     buf(nxt), gsem)

                pltpu.make_async_copy(hs_hbm.at[idx_ref.at[g]], buf(par),
                                      gsem).wait()
                pltpu.async_copy(buf(par), acc.at[dst_sl.at[g]], ssem,
                                 add=True)
                return 0

            lax.fori_loop(0, gpt, sloop, 0)
            pltpu.make_async_copy(buf(lax.bitwise_and(gpt - 1, RING - 1)),
                                  acc.at[dst_sl.at[gpt - 1]], ssem).wait()
            plsc.subcore_barrier()

            pltpu.sync_copy(acc.at[pl.ds(base, ROWS_PER_TILE)],
                            out_hbm.at[out_slot].at[pl.ds(base, ROWS_PER_TILE)])

            if k + 1 < chunks_per_sc:
                plsc.subcore_barrier()
                zero_acc()
                plsc.subcore_barrier()

    return kern(hs2, src2, dst2)


def _dinv_of(deg_ref):
    d = deg_ref[0, :, 0:1] + deg_ref[1, :, 0:1]
    return lax.rsqrt(jnp.maximum(d, 1.0))


def _pre_body(deg_ref, x_ref, out_ref):
    out_ref[...] = x_ref[...] * _dinv_of(deg_ref)


def _mm1_body(deg_ref, p_ref, w_ref, out_ref):
    dinv = _dinv_of(deg_ref)
    t = jnp.zeros((RB, HID), jnp.float32)
    for c in range(IN // 64):
        t += jnp.dot(p_ref[c], w_ref[pl.ds(c * 64, 64), :],
                     preferred_element_type=jnp.float32)
    out_ref[...] = dinv * jax.nn.relu(dinv * t)


def _mm23_body(deg_ref, p_ref, w1_ref, w2_ref, out_ref):
    dinv = _dinv_of(deg_ref)
    t = jnp.zeros((RB, HID), jnp.float32)
    for c in range(HID // 64):
        t += jnp.dot(p_ref[c], w1_ref[pl.ds(c * 64, 64), :],
                     preferred_element_type=jnp.float32)
    h2 = jax.nn.relu(dinv * t)
    out_ref[...] = dinv * jnp.dot(h2, w2_ref[...],
                                  preferred_element_type=jnp.float32)


def _final_body(deg_ref, p_ref, out_ref):
    dinv = _dinv_of(deg_ref)
    y = dinv * (p_ref[0] + p_ref[1])
    m = jnp.max(y, axis=1, keepdims=True)
    e = jnp.exp(y - m)
    out_ref[...] = (y - m) - jnp.log(jnp.sum(e, axis=1, keepdims=True))


def _deg_spec():
    return pl.BlockSpec((2, RB, 16), lambda i: (0, i, 0))


def _chunk_spec(nc, width=128):
    return pl.BlockSpec((nc, RB, width), lambda i: (0, i, 0))


def _full_spec(shape):
    return pl.BlockSpec(shape, lambda i: tuple(0 for _ in shape))


def _row_spec(width):
    return pl.BlockSpec((RB, width), lambda i: (i, 0))


_GRID = (N // RB,)


def _pre_tc(deg2, x):
    return pl.pallas_call(
        _pre_body,
        grid=_GRID,
        in_specs=[_deg_spec(), _row_spec(IN)],
        out_specs=_row_spec(IN),
        out_shape=jax.ShapeDtypeStruct((N, IN), jnp.float32),
    )(deg2, x)


def _mm1_tc(deg2, p1, w0):
    return pl.pallas_call(
        _mm1_body,
        grid=_GRID,
        in_specs=[_deg_spec(), _chunk_spec(4, 64), _full_spec((IN, HID))],
        out_specs=_row_spec(HID),
        out_shape=jax.ShapeDtypeStruct((N, HID), jnp.float32),
    )(deg2, p1, w0)


def _mm23_tc(deg2, p2, w1, w2):
    return pl.pallas_call(
        _mm23_body,
        grid=_GRID,
        in_specs=[_deg_spec(), _chunk_spec(8, 64), _full_spec((HID, HID)),
                  _full_spec((HID, CLS))],
        out_specs=_row_spec(CLS),
        out_shape=jax.ShapeDtypeStruct((N, CLS), jnp.float32),
    )(deg2, p2, w1, w2)


def _final_tc(deg2, p3):
    return pl.pallas_call(
        _final_body,
        grid=_GRID,
        in_specs=[_deg_spec(), _chunk_spec(2, CLS)],
        out_specs=_row_spec(CLS),
        out_shape=jax.ShapeDtypeStruct((N, CLS), jnp.float32),
    )(deg2, p3)


def kernel(inputs, edge_index, W0, W1, W2):
    src = edge_index[0]
    dst = edge_index[1]
    pad = E_PAD - E
    src_p = jnp.concatenate([src, jnp.zeros((pad,), jnp.int32)])
    dst_p = jnp.concatenate([dst, jnp.full((pad,), N, jnp.int32)])
    src2 = src_p.reshape(E_PAD // G, G)
    dst2 = dst_p.reshape(E_PAD // G, G)

    deg2 = _deg_sc(dst2)
    xs = _pre_tc(deg2, inputs)
    p1 = _propagate_sc(xs.reshape(N * 4, 64), src2, dst2,
                       nc=4, width=64, split_edges=False)
    hs1 = _mm1_tc(deg2, p1, W0)
    p2 = _propagate_sc(hs1.reshape(N * 8, 64), src2, dst2,
                       nc=8, width=64, split_edges=False)
    g = _mm23_tc(deg2, p2, W1, W2)
    p3 = _propagate_sc(g, src2, dst2,
                       nc=1, width=CLS, split_edges=True)
    return _final_tc(deg2, p3)


# speedup vs baseline: 6.1839x; 1.0005x over previous
import functools

import jax
import jax.numpy as jnp
from jax import lax
from jax.experimental import pallas as pl
from jax.experimental.pallas import tpu as pltpu
from jax.experimental.pallas import tpu_sc as plsc

N = 10000
E = 160000
IN = 256
HID = 512
CLS = 64

N_PAD = 10240
G = 128
E_ROWS = 1280
NSC = 2
NT = 16
ROWS_PER_TILE = N_PAD // NT
RB = 2000
RING = 4

_mesh = plsc.VectorSubcoreMesh(core_axis_name="c", subcore_axis_name="s")
_SC_PARAMS = pltpu.CompilerParams(use_tc_tiling_on_sc=False)


def _zero_fill(buf, width):
    zv = jnp.zeros((16,), jnp.float32)

    def row(g, _):
        for i in range(width // 16):
            buf[g, pl.ds(i * 16, 16)] = zv
        return 0

    lax.fori_loop(0, G, row, 0)


def _ring_stream(hs_hbm, idx_ref, dst_sl, rows, acc, gsem, ssem, gpt, width):
    def buf(par):
        return rows.at[pl.ds(par * G, G)]

    for i in range(RING - 1):
        pltpu.async_copy(hs_hbm.at[idx_ref.at[i]], buf(i), gsem)

    def sloop(g, _):
        par = lax.bitwise_and(g, RING - 1)
        nxt = lax.bitwise_and(g + RING - 1, RING - 1)

        @pl.when(g >= 1)
        def _():
            pltpu.make_async_copy(buf(nxt), acc.at[dst_sl.at[g - 1]],
                                  ssem).wait()

        @pl.when(g + RING - 1 < gpt)
        def _():
            pltpu.async_copy(hs_hbm.at[idx_ref.at[g + RING - 1]], buf(nxt),
                             gsem)

        pltpu.make_async_copy(hs_hbm.at[idx_ref.at[g]], buf(par),
                              gsem).wait()
        pltpu.async_copy(buf(par), acc.at[dst_sl.at[g]], ssem, add=True)
        return 0

    lax.fori_loop(0, gpt, sloop, 0)
    pltpu.make_async_copy(buf(0), acc.at[dst_sl.at[0]], ssem).wait()


def _deg_sc(dst2):
    gpt = E_ROWS // (NSC * NT)

    @functools.partial(
        pl.kernel,
        out_type=jax.ShapeDtypeStruct((NSC, N_PAD, 16), jnp.float32),
        mesh=_mesh,
        compiler_params=_SC_PARAMS,
        scratch_types=[
            pltpu.VMEM((gpt, G), jnp.int32),
            pltpu.VMEM((G, 16), jnp.float32),
            pltpu.VMEM((G, 16), jnp.float32),
            pltpu.VMEM_SHARED((N_PAD, 16), jnp.float32),
        ],
    )
    def kern(dst2_hbm, out_hbm, dst_sl, ones_b, tmp_b, acc):
        cid = lax.axis_index("c")
        sid = lax.axis_index("s")
        wid = cid * NT + sid
        pltpu.sync_copy(dst2_hbm.at[pl.ds(wid * gpt, gpt)], dst_sl)

        ov = jnp.ones((16,), jnp.float32)

        def fill(g, _):
            ones_b[g] = ov
            return 0

        lax.fori_loop(0, G, fill, 0)
        _zero_fill(tmp_b, 16)

        base = sid * ROWS_PER_TILE
        for b in range(ROWS_PER_TILE // G):
            pltpu.sync_copy(tmp_b, acc.at[pl.ds(base + b * G, G)])
        plsc.subcore_barrier()

        def sloop(g, _):
            pltpu.sync_copy(ones_b, acc.at[dst_sl.at[g]], add=True)
            return 0

        lax.fori_loop(0, gpt, sloop, 0)
        plsc.subcore_barrier()

        pltpu.sync_copy(acc.at[pl.ds(base, ROWS_PER_TILE)],
                        out_hbm.at[cid].at[pl.ds(base, ROWS_PER_TILE)])

    return kern(dst2)


def _propagate_sc(hs2, src2, dst2, nc, split_edges):
    width = 64
    if split_edges:
        assert nc == 1
        n_out = NSC
        chunks_per_sc = 1
        gpt = E_ROWS // 2 // NT
    else:
        n_out = nc
        chunks_per_sc = nc // NSC
        gpt = E_ROWS // NT

    @functools.partial(
        pl.kernel,
        out_type=jax.ShapeDtypeStruct((n_out, N_PAD, width), jnp.float32),
        mesh=_mesh,
        compiler_params=_SC_PARAMS,
        scratch_types=[
            pltpu.VMEM((gpt, G), jnp.int32),
            pltpu.VMEM((gpt, G), jnp.int32),
            pltpu.VMEM((gpt, G), jnp.int32),
            pltpu.VMEM((RING * G, width), jnp.float32),
            pltpu.VMEM((G, width), jnp.float32),
            pltpu.VMEM_SHARED((N_PAD, width), jnp.float32),
            pltpu.SemaphoreType.DMA,
            pltpu.SemaphoreType.DMA,
        ],
    )
    def kern(hs_hbm, src2_hbm, dst2_hbm, out_hbm,
             src_sl, dst_sl, gidx, rows, zbuf, acc, gsem, ssem):
        cid = lax.axis_index("c")
        sid = lax.axis_index("s")
        if split_edges:
            row0 = (cid * NT + sid) * gpt
        else:
            row0 = sid * gpt
        pltpu.sync_copy(src2_hbm.at[pl.ds(row0, gpt)], src_sl)
        pltpu.sync_copy(dst2_hbm.at[pl.ds(row0, gpt)], dst_sl)

        _zero_fill(zbuf, width)
        base = sid * ROWS_PER_TILE

        def zero_acc():
            for b in range(ROWS_PER_TILE // G):
                pltpu.sync_copy(zbuf, acc.at[pl.ds(base + b * G, G)])

        zero_acc()
        plsc.subcore_barrier()

        for k in range(chunks_per_sc):
            chunk = cid * chunks_per_sc + k
            out_slot = cid if split_edges else chunk

            if nc == 1:
                idx_ref = src_sl
            else:
                idx_ref = gidx

                def bidx(g, _):
                    for i in range(G // 16):
                        v = src_sl[g, pl.ds(i * 16, 16)]
                        gidx[g, pl.ds(i * 16, 16)] = v * nc + chunk
                    return 0

                lax.fori_loop(0, gpt, bidx, 0)

            _ring_stream(hs_hbm, idx_ref, dst_sl, rows, acc, gsem, ssem,
                         gpt, width)
            plsc.subcore_barrier()

            pltpu.sync_copy(acc.at[pl.ds(base, ROWS_PER_TILE)],
                            out_hbm.at[out_slot].at[pl.ds(base,
                                                          ROWS_PER_TILE)])

            if k + 1 < chunks_per_sc:
                zero_acc()
                plsc.subcore_barrier()

    return kern(hs2, src2, dst2)


def _dinv_of(deg_ref):
    d = deg_ref[0, :, 0:1] + deg_ref[1, :, 0:1]
    return lax.rsqrt(jnp.maximum(d, 1.0))


def _pre_body(deg_ref, x_ref, out_ref):
    out_ref[...] = x_ref[...] * _dinv_of(deg_ref)


def _mm1_body(deg_ref, p_ref, w_ref, out_ref):
    dinv = _dinv_of(deg_ref)
    t = jnp.zeros((RB, HID), jnp.float32)
    for c in range(IN // 64):
        t += jnp.dot(p_ref[c], w_ref[pl.ds(c * 64, 64), :],
                     preferred_element_type=jnp.float32)
    out_ref[...] = dinv * jax.nn.relu(dinv * t)


def _mm23_body(deg_ref, p_ref, w1_ref, w2_ref, out_ref):
    dinv = _dinv_of(deg_ref)
    t = jnp.zeros((RB, HID), jnp.float32)
    for c in range(HID // 64):
        t += jnp.dot(p_ref[c], w1_ref[pl.ds(c * 64, 64), :],
                     preferred_element_type=jnp.float32)
    h2 = jax.nn.relu(dinv * t)
    out_ref[...] = dinv * jnp.dot(h2, w2_ref[...],
                                  preferred_element_type=jnp.float32)


def _final_body(deg_ref, p_ref, out_ref):
    dinv = _dinv_of(deg_ref)
    y = dinv * (p_ref[0] + p_ref[1])
    m = jnp.max(y, axis=1, keepdims=True)
    e = jnp.exp(y - m)
    out_ref[...] = (y - m) - jnp.log(jnp.sum(e, axis=1, keepdims=True))


def _deg_spec():
    return pl.BlockSpec((2, RB, 16), lambda i: (0, i, 0))


def _chunk_spec(nc, width):
    return pl.BlockSpec((nc, RB, width), lambda i: (0, i, 0))


def _full_spec(shape):
    return pl.BlockSpec(shape, lambda i: tuple(0 for _ in shape))


def _row_spec(width):
    return pl.BlockSpec((RB, width), lambda i: (i, 0))


_GRID = (N // RB,)


def _pre_tc(deg2, x):
    return pl.pallas_call(
        _pre_body,
        grid=_GRID,
        in_specs=[_deg_spec(), _row_spec(IN)],
        out_specs=_row_spec(IN),
        out_shape=jax.ShapeDtypeStruct((N, IN), jnp.float32),
    )(deg2, x)


def _mm1_tc(deg2, p1, w0):
    return pl.pallas_call(
        _mm1_body,
        grid=_GRID,
        in_specs=[_deg_spec(), _chunk_spec(4, 64), _full_spec((IN, HID))],
        out_specs=_row_spec(HID),
        out_shape=jax.ShapeDtypeStruct((N, HID), jnp.float32),
    )(deg2, p1, w0)


def _mm23_tc(deg2, p2, w1, w2):
    return pl.pallas_call(
        _mm23_body,
        grid=_GRID,
        in_specs=[_deg_spec(), _chunk_spec(8, 64), _full_spec((HID, HID)),
                  _full_spec((HID, CLS))],
        out_specs=_row_spec(CLS),
        out_shape=jax.ShapeDtypeStruct((N, CLS), jnp.float32),
    )(deg2, p2, w1, w2)


def _final_tc(deg2, p3):
    return pl.pallas_call(
        _final_body,
        grid=_GRID,
        in_specs=[_deg_spec(), _chunk_spec(2, CLS)],
        out_specs=_row_spec(CLS),
        out_shape=jax.ShapeDtypeStruct((N, CLS), jnp.float32),
    )(deg2, p3)


def kernel(inputs, edge_index, W0, W1, W2):
    src = edge_index[0]
    dst = edge_index[1]
    pad = E_ROWS * G - E
    src_p = jnp.concatenate([src, jnp.zeros((pad,), jnp.int32)])
    dst_p = jnp.concatenate([dst, jnp.full((pad,), N, jnp.int32)])
    src2 = src_p.reshape(E_ROWS, G)
    dst2 = dst_p.reshape(E_ROWS, G)

    deg2 = _deg_sc(dst2)
    xs = _pre_tc(deg2, inputs)
    p1 = _propagate_sc(xs.reshape(N * 4, 64), src2, dst2,
                       nc=4, split_edges=False)
    hs1 = _mm1_tc(deg2, p1, W0)
    p2 = _propagate_sc(hs1.reshape(N * 8, 64), src2, dst2,
                       nc=8, split_edges=False)
    g = _mm23_tc(deg2, p2, W1, W2)
    p3 = _propagate_sc(g, src2, dst2,
                       nc=1, split_edges=True)
    return _final_tc(deg2, p3)
